# baseline (speedup 1.0000x reference)
import functools

import jax
import jax.numpy as jnp
from jax import lax
from jax.experimental import pallas as pl
from jax.experimental.pallas import tpu as pltpu
from jax.experimental.pallas import tpu_sc as plsc

CHUNK = 128


@functools.cache
def _make_gather2(B, D):
    info = plsc.get_sparse_core_info()
    nw = info.num_cores * info.num_subcores
    rows_per_worker = B // nw
    n_chunks = rows_per_worker // CHUNK
    mesh = plsc.VectorSubcoreMesh(core_axis_name="c", subcore_axis_name="s")

    @functools.partial(
        pl.kernel,
        mesh=mesh,
        compiler_params=pltpu.CompilerParams(use_tc_tiling_on_sc=False),
        out_type=[
            jax.ShapeDtypeStruct((B, D), jnp.float32),
            jax.ShapeDtypeStruct((B, D), jnp.float32),
        ],
        scratch_types=[
            pltpu.VMEM((CHUNK,), jnp.int32),
            pltpu.VMEM((CHUNK, D), jnp.float32),
            pltpu.VMEM((CHUNK,), jnp.int32),
            pltpu.VMEM((CHUNK, D), jnp.float32),
            pltpu.SemaphoreType.DMA,
            pltpu.SemaphoreType.DMA,
        ],
    )
    def gather2(qi_hbm, vi_hbm, qt_hbm, kt_hbm, qo_hbm, vo_hbm,
                qidx_v, qrows_v, vidx_v, vrows_v, qsem, vsem):
        wid = lax.axis_index("s") * info.num_cores + lax.axis_index("c")
        base = wid * rows_per_worker

        def step(i, carry):
            off = base + i * CHUNK
            pltpu.sync_copy(qi_hbm.at[pl.ds(off, CHUNK)], qidx_v)
            pltpu.sync_copy(vi_hbm.at[pl.ds(off, CHUNK)], vidx_v)
            qcp = pltpu.async_copy(qt_hbm.at[qidx_v], qrows_v, qsem)
            vcp = pltpu.async_copy(kt_hbm.at[vidx_v], vrows_v, vsem)
            qcp.wait()
            pltpu.sync_copy(qrows_v, qo_hbm.at[pl.ds(off, CHUNK)])
            vcp.wait()
            pltpu.sync_copy(vrows_v, vo_hbm.at[pl.ds(off, CHUNK)])
            return carry

        lax.fori_loop(0, n_chunks, step, 0)

    return gather2


def kernel(queries, values, query_table, key_table):
    batch, hist = queries.shape
    depth = query_table.shape[1]
    flat = batch * hist
    qi = queries.reshape(flat).astype(jnp.int32)
    vi = values.reshape(flat).astype(jnp.int32)
    q_out, v_out = _make_gather2(flat, depth)(qi, vi, query_table, key_table)
    return q_out.reshape(batch, hist, depth), v_out.reshape(batch, hist, depth)


# speedup vs baseline: 1.1385x; 1.1385x over previous
import functools

import jax
import jax.numpy as jnp
from jax import lax
from jax.experimental import pallas as pl
from jax.experimental.pallas import tpu as pltpu
from jax.experimental.pallas import tpu_sc as plsc

CHUNK = 128
GROUP = 640


@functools.cache
def _make_gather2(B, D):
    info = plsc.get_sparse_core_info()
    nw = info.num_cores * info.num_subcores
    rows_per_worker = B // nw
    n_pairs = rows_per_worker // (2 * GROUP)
    gpb = GROUP // CHUNK
    mesh = plsc.VectorSubcoreMesh(core_axis_name="c", subcore_axis_name="s")

    @functools.partial(
        pl.kernel,
        mesh=mesh,
        compiler_params=pltpu.CompilerParams(use_tc_tiling_on_sc=False),
        out_type=[
            jax.ShapeDtypeStruct((B, D), jnp.float32),
            jax.ShapeDtypeStruct((B, D), jnp.float32),
        ],
        scratch_types=[
            pltpu.VMEM((GROUP,), jnp.int32),
            pltpu.VMEM((GROUP,), jnp.int32),
            pltpu.VMEM((GROUP, D), jnp.float32),
            pltpu.VMEM((GROUP, D), jnp.float32),
            pltpu.SemaphoreType.DMA,
            pltpu.SemaphoreType.DMA,
            pltpu.SemaphoreType.DMA,
            pltpu.SemaphoreType.DMA,
        ],
    )
    def gather2(qi_hbm, vi_hbm, qt_hbm, kt_hbm, qo_hbm, vo_hbm,
                idx0, idx1, rows0, rows1, sg0, sg1, sw0, sw1):
        wid = lax.axis_index("s") * info.num_cores + lax.axis_index("c")
        base = wid * rows_per_worker

        def run_table(idx_hbm, tab_hbm, out_hbm):
            def fill(idx_v, rows_v, gsem, off):
                pltpu.sync_copy(idx_hbm.at[pl.ds(off, GROUP)], idx_v)
                return [
                    pltpu.async_copy(
                        tab_hbm.at[idx_v.at[pl.ds(j * CHUNK, CHUNK)]],
                        rows_v.at[pl.ds(j * CHUNK, CHUNK)],
                        gsem,
                    )
                    for j in range(gpb)
                ]

            def step(i, carry):
                off0 = base + (2 * i) * GROUP
                off1 = off0 + GROUP

                @pl.when(i > 0)
                def _():
                    pltpu.make_async_copy(
                        rows0, out_hbm.at[pl.ds(off0, GROUP)], sw0).wait()

                cps0 = fill(idx0, rows0, sg0, off0)

                @pl.when(i > 0)
                def _():
                    pltpu.make_async_copy(
                        rows1, out_hbm.at[pl.ds(off1, GROUP)], sw1).wait()

                cps1 = fill(idx1, rows1, sg1, off1)

                for c in cps0:
                    c.wait()
                pltpu.async_copy(rows0, out_hbm.at[pl.ds(off0, GROUP)], sw0)
                for c in cps1:
                    c.wait()
                pltpu.async_copy(rows1, out_hbm.at[pl.ds(off1, GROUP)], sw1)
                return carry

            lax.fori_loop(0, n_pairs, step, 0)
            pltpu.make_async_copy(rows0, out_hbm.at[pl.ds(base, GROUP)], sw0).wait()
            pltpu.make_async_copy(rows1, out_hbm.at[pl.ds(base, GROUP)], sw1).wait()

        run_table(qi_hbm, qt_hbm, qo_hbm)
        run_table(vi_hbm, kt_hbm, vo_hbm)

    return gather2


def kernel(queries, values, query_table, key_table):
    batch, hist = queries.shape
    depth = query_table.shape[1]
    flat = batch * hist
    qi = queries.reshape(flat).astype(jnp.int32)
    vi = values.reshape(flat).astype(jnp.int32)
    q_out, v_out = _make_gather2(flat, depth)(qi, vi, query_table, key_table)
    return q_out.reshape(batch, hist, depth), v_out.reshape(batch, hist, depth)


# speedup vs baseline: 1.1861x; 1.0418x over previous
import functools

import jax
import jax.numpy as jnp
from jax import lax
from jax.experimental import pallas as pl
from jax.experimental.pallas import tpu as pltpu
from jax.experimental.pallas import tpu_sc as plsc

CHUNK = 128
GROUP = 640


@functools.cache
def _make_gather(B, D):
    info = plsc.get_sparse_core_info()
    nw = info.num_cores * info.num_subcores
    rows_per_worker = B // nw
    n_pairs = rows_per_worker // (2 * GROUP)
    gpb = GROUP // CHUNK
    mesh = plsc.VectorSubcoreMesh(core_axis_name="c", subcore_axis_name="s")

    @functools.partial(
        pl.kernel,
        mesh=mesh,
        compiler_params=pltpu.CompilerParams(use_tc_tiling_on_sc=False),
        out_type=jax.ShapeDtypeStruct((B, D), jnp.float32),
        scratch_types=[
            pltpu.VMEM((GROUP,), jnp.int32),
            pltpu.VMEM((GROUP,), jnp.int32),
            pltpu.VMEM((GROUP, D), jnp.float32),
            pltpu.VMEM((GROUP, D), jnp.float32),
            pltpu.SemaphoreType.DMA,
            pltpu.SemaphoreType.DMA,
            pltpu.SemaphoreType.DMA,
            pltpu.SemaphoreType.DMA,
        ],
    )
    def gather1(idx_hbm, tab_hbm, out_hbm,
                idx0, idx1, rows0, rows1, sg0, sg1, sw0, sw1):
        wid = lax.axis_index("s") * info.num_cores + lax.axis_index("c")
        base = wid * rows_per_worker

        def fill(idx_v, rows_v, gsem, off):
            pltpu.sync_copy(idx_hbm.at[pl.ds(off, GROUP)], idx_v)
            return [
                pltpu.async_copy(
                    tab_hbm.at[idx_v.at[pl.ds(j * CHUNK, CHUNK)]],
                    rows_v.at[pl.ds(j * CHUNK, CHUNK)],
                    gsem,
                )
                for j in range(gpb)
            ]

        def step(i, carry):
            off0 = base + (2 * i) * GROUP
            off1 = off0 + GROUP

            @pl.when(i > 0)
            def _():
                pltpu.make_async_copy(
                    rows0, out_hbm.at[pl.ds(off0, GROUP)], sw0).wait()

            cps0 = fill(idx0, rows0, sg0, off0)

            @pl.when(i > 0)
            def _():
                pltpu.make_async_copy(
                    rows1, out_hbm.at[pl.ds(off1, GROUP)], sw1).wait()

            cps1 = fill(idx1, rows1, sg1, off1)

            for c in cps0:
                c.wait()
            pltpu.async_copy(rows0, out_hbm.at[pl.ds(off0, GROUP)], sw0)
            for c in cps1:
                c.wait()
            pltpu.async_copy(rows1, out_hbm.at[pl.ds(off1, GROUP)], sw1)
            return carry

        lax.fori_loop(0, n_pairs, step, 0)
        pltpu.make_async_copy(rows0, out_hbm.at[pl.ds(base, GROUP)], sw0).wait()
        pltpu.make_async_copy(rows1, out_hbm.at[pl.ds(base, GROUP)], sw1).wait()

    return gather1


def kernel(queries, values, query_table, key_table):
    batch, hist = queries.shape
    depth = query_table.shape[1]
    flat = batch * hist
    gather = _make_gather(flat, depth)
    q_out = gather(queries.reshape(flat).astype(jnp.int32), query_table)
    v_out = gather(values.reshape(flat).astype(jnp.int32), key_table)
    return q_out.reshape(batch, hist, depth), v_out.reshape(batch, hist, depth)
